# baseline (speedup 1.0000x reference)
import functools

import numpy as np
import jax
import jax.numpy as jnp
from jax import lax
from jax.experimental import pallas as pl
from jax.experimental.pallas import tpu as pltpu
from jax.experimental.pallas import tpu_sc as plsc

_H = 256
_W = 256
_N = 1000
_NP = 1024
_NWORDS = _NP // 32


def _iota16():
    return lax.iota(jnp.int32, 16)


def _full16(v):
    return jnp.full((16,), v, jnp.int32)


def _srl(x, n):
    return lax.shift_right_logical(x, n)


def _bitc(b):
    return jnp.int32(np.int32(np.uint32(1 << b)))


def _msb_index(v):
    neg = v < 0
    u = v
    u = u | _srl(u, 1)
    u = u | _srl(u, 2)
    u = u | _srl(u, 4)
    u = u | _srl(u, 8)
    u = u | _srl(u, 16)
    iso = u ^ _srl(u, 1)
    eb = _srl(lax.bitcast_convert_type(iso.astype(jnp.float32),
                                       jnp.int32), 23) - 127
    return jnp.where(neg, 31, eb)


def _sc_body(boxes_hbm, out_hbm, bx_v, py1_v, px1_v, py2_v, px2_v,
             rw_v, cw_part, cw_v, outb, cw_sh):
    cid = lax.axis_index("c")
    sid = lax.axis_index("s")
    wid = cid * 16 + sid

    pltpu.sync_copy(boxes_hbm, bx_v)

    lanes = _iota16()

    def cvt(g, carry):
        for u in range(4):
            base = g * 64 + u * 16
            bi = base + lanes
            bic = jnp.minimum(bi, _N - 1)
            valid = bi < _N
            bic4 = bic * 4
            b0 = plsc.load_gather(bx_v, [bic4])
            b1 = plsc.load_gather(bx_v, [bic4 + 1])
            b2 = plsc.load_gather(bx_v, [bic4 + 2])
            b3 = plsc.load_gather(bx_v, [bic4 + 3])
            sl = pl.ds(base, 16)
            py1_v[sl] = jnp.maximum(0, (b0 * _H).astype(jnp.int32))
            px1_v[sl] = jnp.maximum(0, (b1 * _W).astype(jnp.int32))
            py2_v[sl] = jnp.where(
                valid, jnp.minimum(_H, (b2 * _H).astype(jnp.int32)), 0)
            px2_v[sl] = jnp.where(
                valid, jnp.minimum(_W, (b3 * _W).astype(jnp.int32)), 0)
        return carry

    lax.fori_loop(0, _NP // 64, cvt, 0)

    idx_lo = lanes * 32
    idx_hi = idx_lo + 512

    y0 = wid * 8

    def rw_bit(b, accs):
        y1lo = plsc.load_gather(py1_v, [idx_lo + b])
        y2lo = plsc.load_gather(py2_v, [idx_lo + b])
        y1hi = plsc.load_gather(py1_v, [idx_hi + b])
        y2hi = plsc.load_gather(py2_v, [idx_hi + b])
        bit = jnp.int32(1) << b
        out = []
        for r in range(8):
            wlo, whi = accs[r]
            y = y0 + r
            mlo = (y >= y1lo) & (y < y2lo)
            mhi = (y >= y1hi) & (y < y2hi)
            out.append((wlo | jnp.where(mlo, bit, 0),
                        whi | jnp.where(mhi, bit, 0)))
        return tuple(out)

    z = jnp.zeros((16,), jnp.int32)
    accs = lax.fori_loop(0, 32, rw_bit, tuple((z, z) for _ in range(8)))
    for r in range(8):
        rw_v[r, 0:16] = accs[r][0]
        rw_v[r, 16:32] = accs[r][1]

    xsv = lanes + sid * 16

    def cw_word(w, carry):
        wvec0 = jnp.zeros((16,), jnp.int32)
        wvec1 = jnp.zeros((16,), jnp.int32)
        for half in range(2):
            p1v = px1_v[pl.ds(w * 32 + half * 16, 16)]
            p2v = px2_v[pl.ds(w * 32 + half * 16, 16)]
            for b in range(16):
                idxc = _full16(b)
                p1 = p1v.at[idxc].get(mode="promise_in_bounds")
                p2 = p2v.at[idxc].get(mode="promise_in_bounds")
                m = (xsv >= p1) & (xsv < p2)
                bb = half * 16 + b
                if bb % 2:
                    wvec1 = wvec1 | jnp.where(m, _bitc(bb), 0)
                else:
                    wvec0 = wvec0 | jnp.where(m, _bitc(bb), 0)
        cw_part[w, 0:16] = wvec0 | wvec1
        return carry

    lax.fori_loop(0, _NWORDS, cw_word, 0)

    pltpu.sync_copy(cw_part, cw_sh.at[sid])
    plsc.subcore_barrier()
    pltpu.sync_copy(cw_sh, cw_v)

    def row_loop(r, carry):
        rwlo = rw_v[r, 0:16]
        rwhi = rw_v[r, 16:32]

        def scan_words(j, ks, nzm0, nzm1):
            for i, k in enumerate(ks):
                half = rwlo if k < 16 else rwhi
                rk = half.at[_full16(k % 16)].get(mode="promise_in_bounds")
                cw = cw_v[j, k, 0:16]
                nz = (rk & cw) != 0
                if i % 2 == 0:
                    nzm0 = nzm0 | jnp.where(nz, _bitc(k), 0)
                else:
                    nzm1 = nzm1 | jnp.where(nz, _bitc(k), 0)
            return nzm0, nzm1

        z16 = jnp.zeros((16,), jnp.int32)

        def resolve(j, nzm):
            covered = nzm != 0
            kcl = jnp.maximum(_msb_index(nzm), 0)
            rk2 = plsc.load_gather(rw_v, [_full16(r), kcl])
            cw2 = plsc.load_gather(cw_v, [_full16(j), kcl, lanes])
            a2 = rk2 & cw2
            bpos = jnp.maximum(_msb_index(a2), 0)
            idx = jnp.minimum(kcl * 32 + bpos, _N - 1)
            sl = pl.ds(j * 16, 16)
            outb[0, r, sl] = jnp.where(covered, jnp.float32(1.0), 0.0)
            for ch in range(4):
                vc = plsc.load_gather(bx_v, [idx * 4 + ch])
                outb[ch + 1, r, sl] = jnp.where(covered, vc, 0.0)

        def rest(j):
            def go(nzm_in):
                a, b = scan_words(j, range(0, 24), nzm_in, z16)
                return a | b
            return go

        def pair_loop(jj, carry2):
            j0 = jj * 2
            j1 = j0 + 1
            a0, b0 = scan_words(j0, range(24, 32), z16, z16)
            a1, b1 = scan_words(j1, range(24, 32), z16, z16)
            nzm0 = a0 | b0
            nzm1 = a1 | b1
            nzm0 = lax.cond(jnp.all(nzm0 != 0), lambda n: n, rest(j0), nzm0)
            nzm1 = lax.cond(jnp.all(nzm1 != 0), lambda n: n, rest(j1), nzm1)
            resolve(j0, nzm0)
            resolve(j1, nzm1)
            return carry2

        lax.fori_loop(0, 8, pair_loop, 0)
        return carry

    lax.fori_loop(0, 8, row_loop, 0)

    pltpu.sync_copy(outb, out_hbm.at[:, pl.ds(wid * 8, 8), :])


@jax.jit
def kernel(boxes):
    mesh = plsc.VectorSubcoreMesh(core_axis_name="c", subcore_axis_name="s")
    sc = functools.partial(
        pl.kernel,
        mesh=mesh,
        compiler_params=pltpu.CompilerParams(needs_layout_passes=False),
        out_type=jax.ShapeDtypeStruct((5, _H, _W), jnp.float32),
        scratch_types=[
            pltpu.VMEM((_N * 4,), jnp.float32),
            pltpu.VMEM((_NP,), jnp.int32),
            pltpu.VMEM((_NP,), jnp.int32),
            pltpu.VMEM((_NP,), jnp.int32),
            pltpu.VMEM((_NP,), jnp.int32),
            pltpu.VMEM((8, _NWORDS), jnp.int32),
            pltpu.VMEM((_NWORDS, 16), jnp.int32),
            pltpu.VMEM((16, _NWORDS, 16), jnp.int32),
            pltpu.VMEM((5, 8, _W), jnp.float32),
            pltpu.VMEM_SHARED((16, _NWORDS, 16), jnp.int32),
        ],
    )(_sc_body)
    return sc(boxes.reshape(-1))[None]
